# baseline (speedup 1.0000x reference)
import functools

import jax
import jax.numpy as jnp
from jax import lax
from jax.experimental import pallas as pl
from jax.experimental.pallas import tpu as pltpu
from jax.experimental.pallas import tpu_sc as plsc

NUM_ENTS = 10000
NUM_RELS = 230
H = 128
E = 320000
ALPHA = 0.5
PI = 3.141592653589793
SLOPE = (1.0 / 8.0 + 1.0 / 3.0) / 2.0
DEGW = 16

NC = 2
NS = 16
NW = NC * NS
EPW = E // NW
GRP = 40
SCH = 5
CPS = 50
NBUF = 5
DGP = 100
DPS = 20
RPT = 624
ZCH = 208
ZB = 48


def _make_seg_sum():
    mesh = plsc.VectorSubcoreMesh(core_axis_name="c", subcore_axis_name="s",
                                  num_cores=NC, num_subcores=NS)
    scratch = [
        pltpu.VMEM((CPS, GRP), jnp.int32),
        pltpu.VMEM((CPS, GRP), jnp.int32),
        pltpu.VMEM((DPS, DGP), jnp.int32),
        pltpu.VMEM((NBUF, GRP, H), jnp.float32),
        pltpu.VMEM((ZB, H), jnp.float32),
        pltpu.VMEM((ZB, DEGW), jnp.float32),
        pltpu.VMEM((DGP, DEGW), jnp.float32),
        pltpu.VMEM((GRP, DEGW), jnp.float32),
        pltpu.VMEM_SHARED((NUM_ENTS, H), jnp.float32),
        pltpu.VMEM_SHARED((NUM_ENTS, DEGW), jnp.float32),
    ] + [pltpu.SemaphoreType.DMA] * (2 * NBUF + 2)

    @functools.partial(
        pl.kernel,
        out_type=(jax.ShapeDtypeStruct((NC, NUM_ENTS, H), jnp.float32),
                  jax.ShapeDtypeStruct((NC, NUM_ENTS, DEGW), jnp.float32)),
        mesh=mesh,
        compiler_params=pltpu.CompilerParams(use_tc_tiling_on_sc=False),
        scratch_types=scratch,
    )
    def seg_sum(tab, idx4, dst4, dst4d, out, deg_out, idx_v, dst_v, dg_v, ring,
                zbuf, zbuf_d, ones_v, dbuf, acc, deg, *sems):
        gsem = sems[:NBUF]
        dsem = sems[NBUF:2 * NBUF]
        hsem = sems[2 * NBUF:]
        c = lax.axis_index("c")
        s = lax.axis_index("s")
        wid = c * NS + s
        zero16 = jnp.zeros((16,), jnp.float32)
        one16 = jnp.ones((16,), jnp.float32)

        def zero_row(i, carry):
            for j in range(H // 16):
                zbuf[i, pl.ds(j * 16, 16)] = zero16
            zbuf_d[i, pl.ds(0, DEGW)] = zero16
            return carry

        lax.fori_loop(0, ZB, zero_row, 0)

        def ones_row(i, carry):
            ones_v[i, pl.ds(0, DEGW)] = one16
            return carry

        lax.fori_loop(0, DGP, ones_row, 0)

        base = s * RPT
        tail = NUM_ENTS - NS * RPT
        for k in range(RPT // ZB):
            pltpu.async_copy(zbuf, acc.at[pl.ds(base + k * ZB, ZB)], hsem[0])
            pltpu.async_copy(zbuf_d, deg.at[pl.ds(base + k * ZB, ZB)], hsem[1])
        for k in range(RPT // ZB):
            pltpu.make_async_copy(zbuf, acc.at[pl.ds(base + k * ZB, ZB)],
                                  hsem[0]).wait()
            pltpu.make_async_copy(zbuf_d, deg.at[pl.ds(base + k * ZB, ZB)],
                                  hsem[1]).wait()

        @pl.when(s == NS - 1)
        def _():
            pltpu.sync_copy(zbuf.at[pl.ds(0, tail)],
                            acc.at[pl.ds(NS * RPT, tail)])
            pltpu.sync_copy(zbuf_d.at[pl.ds(0, tail)],
                            deg.at[pl.ds(NS * RPT, tail)])

        plsc.subcore_barrier()

        for sc in range(SCH):
            pltpu.sync_copy(idx4.at[wid, sc], idx_v)
            pltpu.sync_copy(dst4.at[wid, sc], dst_v)
            pltpu.sync_copy(dst4d.at[wid, sc], dg_v)
            for b in range(NBUF):
                pltpu.async_copy(tab.at[idx_v.at[b]], ring.at[b], gsem[b])

            def quint(p, carry):
                for q in range(NBUF):
                    g = p * NBUF + q
                    pltpu.make_async_copy(tab.at[idx_v.at[g]], ring.at[q],
                                          gsem[q]).wait()
                    pltpu.sync_copy(ring.at[q], acc.at[dst_v.at[g]], add=True)
                    gf = g + NBUF

                    @pl.when(gf < CPS)
                    def _():
                        pltpu.async_copy(tab.at[idx_v.at[gf]], ring.at[q],
                                         gsem[q])

                @pl.when(p >= 2)
                def _():
                    for q in range(2):
                        pltpu.make_async_copy(
                            ones_v, deg.at[dg_v.at[2 * (p - 2) + q]],
                            dsem[q]).wait()

                for q in range(2):
                    pltpu.async_copy(ones_v, deg.at[dg_v.at[2 * p + q]],
                                     dsem[q], add=True)
                return carry

            lax.fori_loop(0, CPS // NBUF, quint, 0)
            for p in (CPS // NBUF - 2, CPS // NBUF - 1):
                for q in range(2):
                    pltpu.make_async_copy(ones_v, deg.at[dg_v.at[2 * p + q]],
                                          dsem[q]).wait()
        plsc.subcore_barrier()

        chunks = [(base + k * GRP, GRP) for k in range(RPT // GRP)]
        chunks.append((base + (RPT // GRP) * GRP, RPT - (RPT // GRP) * GRP))
        pend = [None, None]
        for k, (r0, n) in enumerate(chunks):
            b = k % 2
            if pend[b] is not None:
                pr, pn = pend[b]
                pltpu.make_async_copy(ring.at[b, pl.ds(0, pn)],
                                      out.at[c, pl.ds(pr, pn)], hsem[b]).wait()
            pltpu.sync_copy(acc.at[pl.ds(r0, n)], ring.at[b, pl.ds(0, n)])
            pltpu.async_copy(ring.at[b, pl.ds(0, n)],
                             out.at[c, pl.ds(r0, n)], hsem[b])
            pend[b] = (r0, n)
        for b in range(2):
            pr, pn = pend[b]
            pltpu.make_async_copy(ring.at[b, pl.ds(0, pn)],
                                  out.at[c, pl.ds(pr, pn)], hsem[b]).wait()
        for r0, n in chunks:
            pltpu.sync_copy(deg.at[pl.ds(r0, n)], dbuf.at[pl.ds(0, n)])
            pltpu.sync_copy(dbuf.at[pl.ds(0, n)], deg_out.at[c, pl.ds(r0, n)])

        @pl.when(s == NS - 1)
        def _():
            pltpu.sync_copy(acc.at[pl.ds(NS * RPT, tail)],
                            ring.at[0, pl.ds(0, tail)])
            pltpu.sync_copy(ring.at[0, pl.ds(0, tail)],
                            out.at[c, pl.ds(NS * RPT, tail)])
            pltpu.sync_copy(deg.at[pl.ds(NS * RPT, tail)],
                            dbuf.at[pl.ds(0, tail)])
            pltpu.sync_copy(dbuf.at[pl.ds(0, tail)],
                            deg_out.at[c, pl.ds(NS * RPT, tail)])

    return seg_sum


_SEG_CACHE = {}


def _seg_sum():
    if "k" not in _SEG_CACHE:
        _SEG_CACHE["k"] = _make_seg_sum()
    return _SEG_CACHE["k"]


def _mm(a, b):
    return jnp.dot(a, b, preferred_element_type=jnp.float32)


def _norm_rows(x):
    n = jnp.sqrt(jnp.sum(x * x, axis=1, keepdims=True))
    return x / jnp.maximum(n, 1e-12)


_RB = 2000


def _stage_a_body(tf_ref, err_ref, rel_ref, st_ref, al_ref, be_ref, twt_ref,
                  twb_ref, wn_ref, lw_ref, ew_ref,
                  related_ref, h0_ref, hw1_ref, lp1_ref, ev1_ref):
    tf = tf_ref[0, 0]
    related_ref[...] = _mm(err_ref[...], rel_ref[...])
    tv = ALPHA * tf * al_ref[...] + (1.0 - ALPHA) * jnp.sin((2.0 * PI * tf) * be_ref[...])
    dyn = _mm(st_ref[...], twt_ref[...]) + _mm(tv, twb_ref[...])
    h0 = _norm_rows(dyn)
    h0_ref[...] = h0
    hw1_ref[...] = _mm(h0, wn_ref[...])
    lp1_ref[...] = _mm(h0, lw_ref[...])
    ev1_ref[...] = _mm(h0, ew_ref[...])


def _stage_a(tf, err_mat, emb_rel, static_emb, alpha_t, beta_t, tw_top, tw_bot,
             w_neigh1, loop_w1, evolve_w1):
    grid = (NUM_ENTS // _RB,)
    row_blk = pl.BlockSpec((_RB, H), lambda i: (i, 0))
    full_sq = pl.BlockSpec((H, H), lambda i: (0, 0))
    out_sd = jax.ShapeDtypeStruct((NUM_ENTS, H), jnp.float32)
    return pl.pallas_call(
        _stage_a_body,
        grid=grid,
        in_specs=[
            pl.BlockSpec(memory_space=pltpu.SMEM),
            pl.BlockSpec((_RB, 2 * NUM_RELS), lambda i: (i, 0)),
            pl.BlockSpec((2 * NUM_RELS, H), lambda i: (0, 0)),
            row_blk, row_blk, row_blk,
            full_sq, full_sq, full_sq, full_sq, full_sq,
        ],
        out_specs=[row_blk] * 5,
        out_shape=[out_sd] * 5,
    )(tf, err_mat, emb_rel, static_emb, alpha_t, beta_t, tw_top, tw_bot,
      w_neigh1, loop_w1, evolve_w1)


def _stage_b_body(racc_ref, dg_ref, acc1_ref, wn1_ref, nn_ref, lp1_ref, ev1_ref,
                  wn2_ref, lw2_ref, ew2_ref,
                  hw2_ref, lm2_ref, rmat_ref):
    rmat = racc_ref[0] + racc_ref[1]
    deg = dg_ref[0, :, 0:1] + dg_ref[1, :, 0:1]
    mask = deg > 0.0
    agg1 = acc1_ref[0] + acc1_ref[1] + _mm(rmat, wn1_ref[...])
    pre = agg1 * nn_ref[...] + jnp.where(mask, lp1_ref[...], ev1_ref[...])
    h1 = jnp.where(pre >= 0.0, pre, SLOPE * pre)
    hw2_ref[...] = _mm(h1, wn2_ref[...])
    lm2_ref[...] = jnp.where(mask, _mm(h1, lw2_ref[...]), _mm(h1, ew2_ref[...]))
    rmat_ref[...] = rmat


def _stage_b(racc, dg, acc1, w_neigh1, node_norm, lp1, ev1, w_neigh2, loop_w2,
             evolve_w2):
    grid = (NUM_ENTS // _RB,)
    row_blk = pl.BlockSpec((_RB, H), lambda i: (i, 0))
    acc_blk = pl.BlockSpec((NC, _RB, H), lambda i: (0, i, 0))
    full_sq = pl.BlockSpec((H, H), lambda i: (0, 0))
    out_sd = jax.ShapeDtypeStruct((NUM_ENTS, H), jnp.float32)
    return pl.pallas_call(
        _stage_b_body,
        grid=grid,
        in_specs=[
            acc_blk,
            pl.BlockSpec((NC, _RB, DEGW), lambda i: (0, i, 0)),
            acc_blk,
            full_sq,
            pl.BlockSpec((_RB, 1), lambda i: (i, 0)),
            row_blk, row_blk,
            full_sq, full_sq, full_sq,
        ],
        out_specs=[row_blk] * 3,
        out_shape=[out_sd] * 3,
    )(racc, dg, acc1, w_neigh1, node_norm, lp1, ev1, w_neigh2, loop_w2,
      evolve_w2)


def _stage_c_body(acc2_ref, rmat_ref, wn2_ref, nn_ref, lm2_ref, h0_ref,
                  related_ref, tgwt_ref, tgb_ref, out_ref):
    agg2 = acc2_ref[0] + acc2_ref[1] + _mm(rmat_ref[...], wn2_ref[...])
    pre = agg2 * nn_ref[...] + lm2_ref[...]
    h2 = jnp.where(pre >= 0.0, pre, SLOPE * pre)
    cur = _norm_rows(h2)
    x = _mm(h0_ref[...] + related_ref[...], tgwt_ref[...]) + tgb_ref[...][None, :]
    ex = jnp.exp(-jnp.abs(x))
    sig = jnp.where(x >= 0.0, 1.0 / (1.0 + ex), ex / (1.0 + ex))
    out = cur * (1.0 - sig) + sig * h0_ref[...]
    out_ref[...] = _norm_rows(out)


def _stage_c(acc2, rmat, w_neigh2, node_norm, lm2, h0, related, tg_wt, tg_b):
    grid = (NUM_ENTS // _RB,)
    row_blk = pl.BlockSpec((_RB, H), lambda i: (i, 0))
    full_sq = pl.BlockSpec((H, H), lambda i: (0, 0))
    return pl.pallas_call(
        _stage_c_body,
        grid=grid,
        in_specs=[
            pl.BlockSpec((NC, _RB, H), lambda i: (0, i, 0)),
            row_blk,
            full_sq,
            pl.BlockSpec((_RB, 1), lambda i: (i, 0)),
            row_blk, row_blk, row_blk,
            full_sq,
            pl.BlockSpec((H,), lambda i: (0,)),
        ],
        out_specs=row_blk,
        out_shape=jax.ShapeDtypeStruct((NUM_ENTS, H), jnp.float32),
    )(acc2, rmat, w_neigh2, node_norm, lm2, h0, related, tg_wt, tg_b)


def kernel(edge_index, edge_type, node_norm, err_mat, t, emb_rel, static_emb,
           alpha_t, beta_t, temporal_w, tg_w, tg_b, w_neigh1, loop_w1,
           evolve_w1, w_neigh2, loop_w2, evolve_w2):
    tf = jnp.asarray(t, jnp.float32).reshape(1, 1)
    src3 = edge_index[0].reshape(NW, SCH, CPS, GRP)
    dst3 = edge_index[1].reshape(NW, SCH, CPS, GRP)
    dst3d = edge_index[1].reshape(NW, SCH, DPS, DGP)
    et3 = edge_type.reshape(NW, SCH, CPS, GRP)
    emb_tab = jnp.zeros((NUM_ENTS, H), jnp.float32).at[:2 * NUM_RELS].set(emb_rel)
    tw_top = temporal_w[:H]
    tw_bot = temporal_w[H:]

    related, h0, hw1, lp1, ev1 = _stage_a(
        tf, err_mat, emb_rel, static_emb, alpha_t, beta_t, tw_top, tw_bot,
        w_neigh1, loop_w1, evolve_w1)
    racc, dg = _seg_sum()(emb_tab, et3, dst3, dst3d)
    acc1, _ = _seg_sum()(hw1, src3, dst3, dst3d)
    hw2, lm2, rmat = _stage_b(
        racc, dg, acc1, w_neigh1, node_norm, lp1, ev1, w_neigh2, loop_w2,
        evolve_w2)
    acc2, _ = _seg_sum()(hw2, src3, dst3, dst3d)
    composed = _stage_c(
        acc2, rmat, w_neigh2, node_norm, lm2, h0, related, tg_w.T, tg_b)
    return (composed, emb_rel)


# speedup vs baseline: 1.0294x; 1.0294x over previous
import functools

import jax
import jax.numpy as jnp
from jax import lax
from jax.experimental import pallas as pl
from jax.experimental.pallas import tpu as pltpu
from jax.experimental.pallas import tpu_sc as plsc

NUM_ENTS = 10000
NUM_RELS = 230
H = 128
E = 320000
ALPHA = 0.5
PI = 3.141592653589793
SLOPE = (1.0 / 8.0 + 1.0 / 3.0) / 2.0
DEGW = 16

NC = 2
NS = 16
NW = NC * NS
EPW = E // NW
GRP = 40
SCH = 5
CPS = 50
NBUF = 5
RPT = 624
ZCH = 208
ZB = 48


def _make_seg_sum():
    mesh = plsc.VectorSubcoreMesh(core_axis_name="c", subcore_axis_name="s",
                                  num_cores=NC, num_subcores=NS)
    scratch = [
        pltpu.VMEM((CPS, GRP), jnp.int32),
        pltpu.VMEM((CPS, GRP), jnp.int32),
        pltpu.VMEM((NBUF, GRP, H), jnp.float32),
        pltpu.VMEM((ZB, H), jnp.float32),
        pltpu.VMEM((ZB, DEGW), jnp.float32),
        pltpu.VMEM((GRP, DEGW), jnp.float32),
        pltpu.VMEM((GRP, DEGW), jnp.float32),
        pltpu.VMEM_SHARED((NUM_ENTS, H), jnp.float32),
        pltpu.VMEM_SHARED((NUM_ENTS, DEGW), jnp.float32),
    ] + [pltpu.SemaphoreType.DMA] * (2 * NBUF + 2)

    @functools.partial(
        pl.kernel,
        out_type=(jax.ShapeDtypeStruct((NC, NUM_ENTS, H), jnp.float32),
                  jax.ShapeDtypeStruct((NC, NUM_ENTS, DEGW), jnp.float32)),
        mesh=mesh,
        compiler_params=pltpu.CompilerParams(use_tc_tiling_on_sc=False),
        scratch_types=scratch,
    )
    def seg_sum(tab, idx4, dst4, out, deg_out, idx_v, dst_v, ring,
                zbuf, zbuf_d, ones_v, dbuf, acc, deg, *sems):
        gsem = sems[:NBUF]
        dsem = sems[NBUF:2 * NBUF]
        hsem = sems[2 * NBUF:]
        c = lax.axis_index("c")
        s = lax.axis_index("s")
        wid = c * NS + s
        zero16 = jnp.zeros((16,), jnp.float32)
        one16 = jnp.ones((16,), jnp.float32)

        def zero_row(i, carry):
            for j in range(H // 16):
                zbuf[i, pl.ds(j * 16, 16)] = zero16
            zbuf_d[i, pl.ds(0, DEGW)] = zero16
            return carry

        lax.fori_loop(0, ZB, zero_row, 0)

        def ones_row(i, carry):
            ones_v[i, pl.ds(0, DEGW)] = one16
            return carry

        lax.fori_loop(0, GRP, ones_row, 0)

        base = s * RPT
        tail = NUM_ENTS - NS * RPT
        for k in range(RPT // ZB):
            pltpu.async_copy(zbuf, acc.at[pl.ds(base + k * ZB, ZB)], hsem[0])
            pltpu.async_copy(zbuf_d, deg.at[pl.ds(base + k * ZB, ZB)], hsem[1])
        for k in range(RPT // ZB):
            pltpu.make_async_copy(zbuf, acc.at[pl.ds(base + k * ZB, ZB)],
                                  hsem[0]).wait()
            pltpu.make_async_copy(zbuf_d, deg.at[pl.ds(base + k * ZB, ZB)],
                                  hsem[1]).wait()

        @pl.when(s == NS - 1)
        def _():
            pltpu.sync_copy(zbuf.at[pl.ds(0, tail)],
                            acc.at[pl.ds(NS * RPT, tail)])
            pltpu.sync_copy(zbuf_d.at[pl.ds(0, tail)],
                            deg.at[pl.ds(NS * RPT, tail)])

        plsc.subcore_barrier()

        for sc in range(SCH):
            pltpu.sync_copy(idx4.at[wid, sc], idx_v)
            pltpu.sync_copy(dst4.at[wid, sc], dst_v)
            for b in range(NBUF):
                pltpu.async_copy(tab.at[idx_v.at[b]], ring.at[b], gsem[b])

            def quint(p, carry):
                for q in range(NBUF):
                    g = p * NBUF + q
                    pltpu.make_async_copy(tab.at[idx_v.at[g]], ring.at[q],
                                          gsem[q]).wait()
                    pltpu.sync_copy(ring.at[q], acc.at[dst_v.at[g]], add=True)

                    @pl.when(g >= NBUF)
                    def _():
                        pltpu.make_async_copy(
                            ones_v, deg.at[dst_v.at[g - NBUF]], dsem[q]).wait()

                    pltpu.async_copy(ones_v, deg.at[dst_v.at[g]], dsem[q],
                                     add=True)
                    gf = g + NBUF

                    @pl.when(gf < CPS)
                    def _():
                        pltpu.async_copy(tab.at[idx_v.at[gf]], ring.at[q],
                                         gsem[q])
                return carry

            lax.fori_loop(0, CPS // NBUF, quint, 0)
            for q in range(NBUF):
                pltpu.make_async_copy(ones_v, deg.at[dst_v.at[CPS - NBUF + q]],
                                      dsem[q]).wait()
        plsc.subcore_barrier()

        chunks = [(base + k * GRP, GRP) for k in range(RPT // GRP)]
        chunks.append((base + (RPT // GRP) * GRP, RPT - (RPT // GRP) * GRP))
        pend = [None, None]
        for k, (r0, n) in enumerate(chunks):
            b = k % 2
            if pend[b] is not None:
                pr, pn = pend[b]
                pltpu.make_async_copy(ring.at[b, pl.ds(0, pn)],
                                      out.at[c, pl.ds(pr, pn)], hsem[b]).wait()
            pltpu.sync_copy(acc.at[pl.ds(r0, n)], ring.at[b, pl.ds(0, n)])
            pltpu.async_copy(ring.at[b, pl.ds(0, n)],
                             out.at[c, pl.ds(r0, n)], hsem[b])
            pend[b] = (r0, n)
        for b in range(2):
            pr, pn = pend[b]
            pltpu.make_async_copy(ring.at[b, pl.ds(0, pn)],
                                  out.at[c, pl.ds(pr, pn)], hsem[b]).wait()
        for r0, n in chunks:
            pltpu.sync_copy(deg.at[pl.ds(r0, n)], dbuf.at[pl.ds(0, n)])
            pltpu.sync_copy(dbuf.at[pl.ds(0, n)], deg_out.at[c, pl.ds(r0, n)])

        @pl.when(s == NS - 1)
        def _():
            pltpu.sync_copy(acc.at[pl.ds(NS * RPT, tail)],
                            ring.at[0, pl.ds(0, tail)])
            pltpu.sync_copy(ring.at[0, pl.ds(0, tail)],
                            out.at[c, pl.ds(NS * RPT, tail)])
            pltpu.sync_copy(deg.at[pl.ds(NS * RPT, tail)],
                            dbuf.at[pl.ds(0, tail)])
            pltpu.sync_copy(dbuf.at[pl.ds(0, tail)],
                            deg_out.at[c, pl.ds(NS * RPT, tail)])

    return seg_sum


_SEG_CACHE = {}


def _seg_sum():
    if "k" not in _SEG_CACHE:
        _SEG_CACHE["k"] = _make_seg_sum()
    return _SEG_CACHE["k"]


def _mm(a, b):
    return jnp.dot(a, b, preferred_element_type=jnp.float32)


def _norm_rows(x):
    n = jnp.sqrt(jnp.sum(x * x, axis=1, keepdims=True))
    return x / jnp.maximum(n, 1e-12)


_RB = 2000


def _stage_a_body(tf_ref, err_ref, rel_ref, st_ref, al_ref, be_ref, twt_ref,
                  twb_ref, wn_ref, lw_ref, ew_ref,
                  related_ref, h0_ref, hw1_ref, lp1_ref, ev1_ref):
    tf = tf_ref[0, 0]
    related_ref[...] = _mm(err_ref[...], rel_ref[...])
    tv = ALPHA * tf * al_ref[...] + (1.0 - ALPHA) * jnp.sin((2.0 * PI * tf) * be_ref[...])
    dyn = _mm(st_ref[...], twt_ref[...]) + _mm(tv, twb_ref[...])
    h0 = _norm_rows(dyn)
    h0_ref[...] = h0
    hw1_ref[...] = _mm(h0, wn_ref[...])
    lp1_ref[...] = _mm(h0, lw_ref[...])
    ev1_ref[...] = _mm(h0, ew_ref[...])


def _stage_a(tf, err_mat, emb_rel, static_emb, alpha_t, beta_t, tw_top, tw_bot,
             w_neigh1, loop_w1, evolve_w1):
    grid = (NUM_ENTS // _RB,)
    row_blk = pl.BlockSpec((_RB, H), lambda i: (i, 0))
    full_sq = pl.BlockSpec((H, H), lambda i: (0, 0))
    out_sd = jax.ShapeDtypeStruct((NUM_ENTS, H), jnp.float32)
    return pl.pallas_call(
        _stage_a_body,
        grid=grid,
        in_specs=[
            pl.BlockSpec(memory_space=pltpu.SMEM),
            pl.BlockSpec((_RB, 2 * NUM_RELS), lambda i: (i, 0)),
            pl.BlockSpec((2 * NUM_RELS, H), lambda i: (0, 0)),
            row_blk, row_blk, row_blk,
            full_sq, full_sq, full_sq, full_sq, full_sq,
        ],
        out_specs=[row_blk] * 5,
        out_shape=[out_sd] * 5,
    )(tf, err_mat, emb_rel, static_emb, alpha_t, beta_t, tw_top, tw_bot,
      w_neigh1, loop_w1, evolve_w1)


def _stage_b_body(racc_ref, dg_ref, acc1_ref, wn1_ref, nn_ref, lp1_ref, ev1_ref,
                  wn2_ref, lw2_ref, ew2_ref,
                  hw2_ref, lm2_ref, rmat_ref):
    rmat = racc_ref[0] + racc_ref[1]
    deg = dg_ref[0, :, 0:1] + dg_ref[1, :, 0:1]
    mask = deg > 0.0
    agg1 = acc1_ref[0] + acc1_ref[1] + _mm(rmat, wn1_ref[...])
    pre = agg1 * nn_ref[...] + jnp.where(mask, lp1_ref[...], ev1_ref[...])
    h1 = jnp.where(pre >= 0.0, pre, SLOPE * pre)
    hw2_ref[...] = _mm(h1, wn2_ref[...])
    lm2_ref[...] = jnp.where(mask, _mm(h1, lw2_ref[...]), _mm(h1, ew2_ref[...]))
    rmat_ref[...] = rmat


def _stage_b(racc, dg, acc1, w_neigh1, node_norm, lp1, ev1, w_neigh2, loop_w2,
             evolve_w2):
    grid = (NUM_ENTS // _RB,)
    row_blk = pl.BlockSpec((_RB, H), lambda i: (i, 0))
    acc_blk = pl.BlockSpec((NC, _RB, H), lambda i: (0, i, 0))
    full_sq = pl.BlockSpec((H, H), lambda i: (0, 0))
    out_sd = jax.ShapeDtypeStruct((NUM_ENTS, H), jnp.float32)
    return pl.pallas_call(
        _stage_b_body,
        grid=grid,
        in_specs=[
            acc_blk,
            pl.BlockSpec((NC, _RB, DEGW), lambda i: (0, i, 0)),
            acc_blk,
            full_sq,
            pl.BlockSpec((_RB, 1), lambda i: (i, 0)),
            row_blk, row_blk,
            full_sq, full_sq, full_sq,
        ],
        out_specs=[row_blk] * 3,
        out_shape=[out_sd] * 3,
    )(racc, dg, acc1, w_neigh1, node_norm, lp1, ev1, w_neigh2, loop_w2,
      evolve_w2)


def _stage_c_body(acc2_ref, rmat_ref, wn2_ref, nn_ref, lm2_ref, h0_ref,
                  related_ref, tgwt_ref, tgb_ref, out_ref):
    agg2 = acc2_ref[0] + acc2_ref[1] + _mm(rmat_ref[...], wn2_ref[...])
    pre = agg2 * nn_ref[...] + lm2_ref[...]
    h2 = jnp.where(pre >= 0.0, pre, SLOPE * pre)
    cur = _norm_rows(h2)
    x = _mm(h0_ref[...] + related_ref[...], tgwt_ref[...]) + tgb_ref[...][None, :]
    ex = jnp.exp(-jnp.abs(x))
    sig = jnp.where(x >= 0.0, 1.0 / (1.0 + ex), ex / (1.0 + ex))
    out = cur * (1.0 - sig) + sig * h0_ref[...]
    out_ref[...] = _norm_rows(out)


def _stage_c(acc2, rmat, w_neigh2, node_norm, lm2, h0, related, tg_wt, tg_b):
    grid = (NUM_ENTS // _RB,)
    row_blk = pl.BlockSpec((_RB, H), lambda i: (i, 0))
    full_sq = pl.BlockSpec((H, H), lambda i: (0, 0))
    return pl.pallas_call(
        _stage_c_body,
        grid=grid,
        in_specs=[
            pl.BlockSpec((NC, _RB, H), lambda i: (0, i, 0)),
            row_blk,
            full_sq,
            pl.BlockSpec((_RB, 1), lambda i: (i, 0)),
            row_blk, row_blk, row_blk,
            full_sq,
            pl.BlockSpec((H,), lambda i: (0,)),
        ],
        out_specs=row_blk,
        out_shape=jax.ShapeDtypeStruct((NUM_ENTS, H), jnp.float32),
    )(acc2, rmat, w_neigh2, node_norm, lm2, h0, related, tg_wt, tg_b)


def kernel(edge_index, edge_type, node_norm, err_mat, t, emb_rel, static_emb,
           alpha_t, beta_t, temporal_w, tg_w, tg_b, w_neigh1, loop_w1,
           evolve_w1, w_neigh2, loop_w2, evolve_w2):
    tf = jnp.asarray(t, jnp.float32).reshape(1, 1)
    src3 = edge_index[0].reshape(NW, SCH, CPS, GRP)
    dst3 = edge_index[1].reshape(NW, SCH, CPS, GRP)
    et3 = edge_type.reshape(NW, SCH, CPS, GRP)
    emb_tab = jnp.zeros((NUM_ENTS, H), jnp.float32).at[:2 * NUM_RELS].set(emb_rel)
    tw_top = temporal_w[:H]
    tw_bot = temporal_w[H:]

    related, h0, hw1, lp1, ev1 = _stage_a(
        tf, err_mat, emb_rel, static_emb, alpha_t, beta_t, tw_top, tw_bot,
        w_neigh1, loop_w1, evolve_w1)
    racc, dg = _seg_sum()(emb_tab, et3, dst3)
    acc1, _ = _seg_sum()(hw1, src3, dst3)
    hw2, lm2, rmat = _stage_b(
        racc, dg, acc1, w_neigh1, node_norm, lp1, ev1, w_neigh2, loop_w2,
        evolve_w2)
    acc2, _ = _seg_sum()(hw2, src3, dst3)
    composed = _stage_c(
        acc2, rmat, w_neigh2, node_norm, lm2, h0, related, tg_w.T, tg_b)
    return (composed, emb_rel)


# speedup vs baseline: 1.1573x; 1.1242x over previous
import functools

import jax
import jax.numpy as jnp
from jax import lax
from jax.experimental import pallas as pl
from jax.experimental.pallas import tpu as pltpu
from jax.experimental.pallas import tpu_sc as plsc

NUM_ENTS = 10000
NUM_RELS = 230
H = 128
E = 320000
ALPHA = 0.5
PI = 3.141592653589793
SLOPE = (1.0 / 8.0 + 1.0 / 3.0) / 2.0
DEGW = 16
REP = 21

NC = 2
NS = 16
NW = NC * NS
EPW = E // NW
GRP = 40
SCH = 5
CPS = 50
NBUF = 5
RPT = 624
ZCH = 208
ZB = 48


def _make_seg_sum():
    mesh = plsc.VectorSubcoreMesh(core_axis_name="c", subcore_axis_name="s",
                                  num_cores=NC, num_subcores=NS)
    scratch = [
        pltpu.VMEM((CPS, GRP), jnp.int32),
        pltpu.VMEM((CPS, GRP), jnp.int32),
        pltpu.VMEM((NBUF, GRP, H), jnp.float32),
        pltpu.VMEM((ZB, H), jnp.float32),
        pltpu.VMEM((ZB, DEGW), jnp.float32),
        pltpu.VMEM((GRP, DEGW), jnp.float32),
        pltpu.VMEM((GRP, DEGW), jnp.float32),
        pltpu.VMEM_SHARED((NUM_ENTS, H), jnp.float32),
        pltpu.VMEM_SHARED((NUM_ENTS, DEGW), jnp.float32),
    ] + [pltpu.SemaphoreType.DMA] * (2 * NBUF + 2)

    @functools.partial(
        pl.kernel,
        out_type=(jax.ShapeDtypeStruct((NC, NUM_ENTS, H), jnp.float32),
                  jax.ShapeDtypeStruct((NC, NUM_ENTS, DEGW), jnp.float32)),
        mesh=mesh,
        compiler_params=pltpu.CompilerParams(use_tc_tiling_on_sc=False),
        scratch_types=scratch,
    )
    def seg_sum(tab, idx4, dst4, out, deg_out, idx_v, dst_v, ring,
                zbuf, zbuf_d, ones_v, dbuf, acc, deg, *sems):
        gsem = sems[:NBUF]
        dsem = sems[NBUF:2 * NBUF]
        hsem = sems[2 * NBUF:]
        c = lax.axis_index("c")
        s = lax.axis_index("s")
        wid = c * NS + s
        zero16 = jnp.zeros((16,), jnp.float32)
        one16 = jnp.ones((16,), jnp.float32)

        def zero_row(i, carry):
            for j in range(H // 16):
                zbuf[i, pl.ds(j * 16, 16)] = zero16
            zbuf_d[i, pl.ds(0, DEGW)] = zero16
            return carry

        lax.fori_loop(0, ZB, zero_row, 0)

        def ones_row(i, carry):
            ones_v[i, pl.ds(0, DEGW)] = one16
            return carry

        lax.fori_loop(0, GRP, ones_row, 0)

        base = s * RPT
        tail = NUM_ENTS - NS * RPT
        for k in range(RPT // ZB):
            pltpu.async_copy(zbuf, acc.at[pl.ds(base + k * ZB, ZB)], hsem[0])
            pltpu.async_copy(zbuf_d, deg.at[pl.ds(base + k * ZB, ZB)], hsem[1])
        for k in range(RPT // ZB):
            pltpu.make_async_copy(zbuf, acc.at[pl.ds(base + k * ZB, ZB)],
                                  hsem[0]).wait()
            pltpu.make_async_copy(zbuf_d, deg.at[pl.ds(base + k * ZB, ZB)],
                                  hsem[1]).wait()

        @pl.when(s == NS - 1)
        def _():
            pltpu.sync_copy(zbuf.at[pl.ds(0, tail)],
                            acc.at[pl.ds(NS * RPT, tail)])
            pltpu.sync_copy(zbuf_d.at[pl.ds(0, tail)],
                            deg.at[pl.ds(NS * RPT, tail)])

        plsc.subcore_barrier()

        for sc in range(SCH):
            pltpu.sync_copy(idx4.at[wid, sc], idx_v)
            pltpu.sync_copy(dst4.at[wid, sc], dst_v)
            for b in range(NBUF):
                pltpu.async_copy(tab.at[idx_v.at[b]], ring.at[b], gsem[b])

            def quint(p, carry):
                for q in range(NBUF):
                    g = p * NBUF + q
                    pltpu.make_async_copy(tab.at[idx_v.at[g]], ring.at[q],
                                          gsem[q]).wait()
                    pltpu.sync_copy(ring.at[q], acc.at[dst_v.at[g]], add=True)

                    @pl.when(g >= NBUF)
                    def _():
                        pltpu.make_async_copy(
                            ones_v, deg.at[dst_v.at[g - NBUF]], dsem[q]).wait()

                    pltpu.async_copy(ones_v, deg.at[dst_v.at[g]], dsem[q],
                                     add=True)
                    gf = g + NBUF

                    @pl.when(gf < CPS)
                    def _():
                        pltpu.async_copy(tab.at[idx_v.at[gf]], ring.at[q],
                                         gsem[q])
                return carry

            lax.fori_loop(0, CPS // NBUF, quint, 0)
            for q in range(NBUF):
                pltpu.make_async_copy(ones_v, deg.at[dst_v.at[CPS - NBUF + q]],
                                      dsem[q]).wait()
        plsc.subcore_barrier()

        chunks = [(base + k * GRP, GRP) for k in range(RPT // GRP)]
        chunks.append((base + (RPT // GRP) * GRP, RPT - (RPT // GRP) * GRP))
        pend = [None, None]
        for k, (r0, n) in enumerate(chunks):
            b = k % 2
            if pend[b] is not None:
                pr, pn = pend[b]
                pltpu.make_async_copy(ring.at[b, pl.ds(0, pn)],
                                      out.at[c, pl.ds(pr, pn)], hsem[b]).wait()
            pltpu.sync_copy(acc.at[pl.ds(r0, n)], ring.at[b, pl.ds(0, n)])
            pltpu.async_copy(ring.at[b, pl.ds(0, n)],
                             out.at[c, pl.ds(r0, n)], hsem[b])
            pend[b] = (r0, n)
        for b in range(2):
            pr, pn = pend[b]
            pltpu.make_async_copy(ring.at[b, pl.ds(0, pn)],
                                  out.at[c, pl.ds(pr, pn)], hsem[b]).wait()
        for r0, n in chunks:
            pltpu.sync_copy(deg.at[pl.ds(r0, n)], dbuf.at[pl.ds(0, n)])
            pltpu.sync_copy(dbuf.at[pl.ds(0, n)], deg_out.at[c, pl.ds(r0, n)])

        @pl.when(s == NS - 1)
        def _():
            pltpu.sync_copy(acc.at[pl.ds(NS * RPT, tail)],
                            ring.at[0, pl.ds(0, tail)])
            pltpu.sync_copy(ring.at[0, pl.ds(0, tail)],
                            out.at[c, pl.ds(NS * RPT, tail)])
            pltpu.sync_copy(deg.at[pl.ds(NS * RPT, tail)],
                            dbuf.at[pl.ds(0, tail)])
            pltpu.sync_copy(dbuf.at[pl.ds(0, tail)],
                            deg_out.at[c, pl.ds(NS * RPT, tail)])

    return seg_sum


_SEG_CACHE = {}


def _seg_sum():
    if "k" not in _SEG_CACHE:
        _SEG_CACHE["k"] = _make_seg_sum()
    return _SEG_CACHE["k"]


def _mm(a, b):
    return jnp.dot(a, b, preferred_element_type=jnp.float32)


def _norm_rows(x):
    n = jnp.sqrt(jnp.sum(x * x, axis=1, keepdims=True))
    return x / jnp.maximum(n, 1e-12)


_RB = 2000


def _stage_a_body(tf_ref, err_ref, rel_ref, st_ref, al_ref, be_ref, twt_ref,
                  twb_ref, wn_ref, lw_ref, ew_ref,
                  related_ref, h0_ref, hw1_ref, lp1_ref, ev1_ref):
    tf = tf_ref[0, 0]
    related_ref[...] = _mm(err_ref[...], rel_ref[...])
    tv = ALPHA * tf * al_ref[...] + (1.0 - ALPHA) * jnp.sin((2.0 * PI * tf) * be_ref[...])
    dyn = _mm(st_ref[...], twt_ref[...]) + _mm(tv, twb_ref[...])
    h0 = _norm_rows(dyn)
    h0_ref[...] = h0
    hw1_ref[...] = _mm(h0, wn_ref[...])
    lp1_ref[...] = _mm(h0, lw_ref[...])
    ev1_ref[...] = _mm(h0, ew_ref[...])


def _stage_a(tf, err_mat, emb_rel, static_emb, alpha_t, beta_t, tw_top, tw_bot,
             w_neigh1, loop_w1, evolve_w1):
    grid = (NUM_ENTS // _RB,)
    row_blk = pl.BlockSpec((_RB, H), lambda i: (i, 0))
    full_sq = pl.BlockSpec((H, H), lambda i: (0, 0))
    out_sd = jax.ShapeDtypeStruct((NUM_ENTS, H), jnp.float32)
    return pl.pallas_call(
        _stage_a_body,
        grid=grid,
        in_specs=[
            pl.BlockSpec(memory_space=pltpu.SMEM),
            pl.BlockSpec((_RB, 2 * NUM_RELS), lambda i: (i, 0)),
            pl.BlockSpec((2 * NUM_RELS, H), lambda i: (0, 0)),
            row_blk, row_blk, row_blk,
            full_sq, full_sq, full_sq, full_sq, full_sq,
        ],
        out_specs=[row_blk] * 5,
        out_shape=[out_sd] * 5,
    )(tf, err_mat, emb_rel, static_emb, alpha_t, beta_t, tw_top, tw_bot,
      w_neigh1, loop_w1, evolve_w1)


def _stage_b_body(racc_ref, dg_ref, acc1_ref, wn1_ref, nn_ref, lp1_ref, ev1_ref,
                  wn2_ref, lw2_ref, ew2_ref,
                  hw2_ref, lm2_ref, rmat_ref):
    rmat = racc_ref[0] + racc_ref[1]
    deg = dg_ref[0, :, 0:1] + dg_ref[1, :, 0:1]
    mask = deg > 0.0
    agg1 = acc1_ref[0] + acc1_ref[1] + _mm(rmat, wn1_ref[...])
    pre = agg1 * nn_ref[...] + jnp.where(mask, lp1_ref[...], ev1_ref[...])
    h1 = jnp.where(pre >= 0.0, pre, SLOPE * pre)
    hw2_ref[...] = _mm(h1, wn2_ref[...])
    lm2_ref[...] = jnp.where(mask, _mm(h1, lw2_ref[...]), _mm(h1, ew2_ref[...]))
    rmat_ref[...] = rmat


def _stage_b(racc, dg, acc1, w_neigh1, node_norm, lp1, ev1, w_neigh2, loop_w2,
             evolve_w2):
    grid = (NUM_ENTS // _RB,)
    row_blk = pl.BlockSpec((_RB, H), lambda i: (i, 0))
    acc_blk = pl.BlockSpec((NC, _RB, H), lambda i: (0, i, 0))
    full_sq = pl.BlockSpec((H, H), lambda i: (0, 0))
    out_sd = jax.ShapeDtypeStruct((NUM_ENTS, H), jnp.float32)
    return pl.pallas_call(
        _stage_b_body,
        grid=grid,
        in_specs=[
            acc_blk,
            pl.BlockSpec((NC, _RB, DEGW), lambda i: (0, i, 0)),
            acc_blk,
            full_sq,
            pl.BlockSpec((_RB, 1), lambda i: (i, 0)),
            row_blk, row_blk,
            full_sq, full_sq, full_sq,
        ],
        out_specs=[row_blk] * 3,
        out_shape=[out_sd] * 3,
    )(racc, dg, acc1, w_neigh1, node_norm, lp1, ev1, w_neigh2, loop_w2,
      evolve_w2)


def _stage_c_body(acc2_ref, rmat_ref, wn2_ref, nn_ref, lm2_ref, h0_ref,
                  related_ref, tgwt_ref, tgb_ref, out_ref):
    agg2 = acc2_ref[0] + acc2_ref[1] + _mm(rmat_ref[...], wn2_ref[...])
    pre = agg2 * nn_ref[...] + lm2_ref[...]
    h2 = jnp.where(pre >= 0.0, pre, SLOPE * pre)
    cur = _norm_rows(h2)
    x = _mm(h0_ref[...] + related_ref[...], tgwt_ref[...]) + tgb_ref[...][None, :]
    ex = jnp.exp(-jnp.abs(x))
    sig = jnp.where(x >= 0.0, 1.0 / (1.0 + ex), ex / (1.0 + ex))
    out = cur * (1.0 - sig) + sig * h0_ref[...]
    out_ref[...] = _norm_rows(out)


def _stage_c(acc2, rmat, w_neigh2, node_norm, lm2, h0, related, tg_wt, tg_b):
    grid = (NUM_ENTS // _RB,)
    row_blk = pl.BlockSpec((_RB, H), lambda i: (i, 0))
    full_sq = pl.BlockSpec((H, H), lambda i: (0, 0))
    return pl.pallas_call(
        _stage_c_body,
        grid=grid,
        in_specs=[
            pl.BlockSpec((NC, _RB, H), lambda i: (0, i, 0)),
            row_blk,
            full_sq,
            pl.BlockSpec((_RB, 1), lambda i: (i, 0)),
            row_blk, row_blk, row_blk,
            full_sq,
            pl.BlockSpec((H,), lambda i: (0,)),
        ],
        out_specs=row_blk,
        out_shape=jax.ShapeDtypeStruct((NUM_ENTS, H), jnp.float32),
    )(acc2, rmat, w_neigh2, node_norm, lm2, h0, related, tg_wt, tg_b)


def kernel(edge_index, edge_type, node_norm, err_mat, t, emb_rel, static_emb,
           alpha_t, beta_t, temporal_w, tg_w, tg_b, w_neigh1, loop_w1,
           evolve_w1, w_neigh2, loop_w2, evolve_w2):
    tf = jnp.asarray(t, jnp.float32).reshape(1, 1)
    rep = 2 * NUM_RELS * (jnp.arange(E, dtype=jnp.int32) % REP)
    et_rep = edge_type + rep
    src3 = edge_index[0].reshape(NW, SCH, CPS, GRP)
    dst3 = edge_index[1].reshape(NW, SCH, CPS, GRP)
    et3 = et_rep.reshape(NW, SCH, CPS, GRP)
    emb_tab = jnp.zeros((NUM_ENTS, H), jnp.float32)
    emb_tab = emb_tab.at[:REP * 2 * NUM_RELS].set(jnp.tile(emb_rel, (REP, 1)))
    tw_top = temporal_w[:H]
    tw_bot = temporal_w[H:]

    related, h0, hw1, lp1, ev1 = _stage_a(
        tf, err_mat, emb_rel, static_emb, alpha_t, beta_t, tw_top, tw_bot,
        w_neigh1, loop_w1, evolve_w1)
    racc, dg = _seg_sum()(emb_tab, et3, dst3)
    acc1, _ = _seg_sum()(hw1, src3, dst3)
    hw2, lm2, rmat = _stage_b(
        racc, dg, acc1, w_neigh1, node_norm, lp1, ev1, w_neigh2, loop_w2,
        evolve_w2)
    acc2, _ = _seg_sum()(hw2, src3, dst3)
    composed = _stage_c(
        acc2, rmat, w_neigh2, node_norm, lm2, h0, related, tg_w.T, tg_b)
    return (composed, emb_rel)
